# baseline (speedup 1.0000x reference)
import functools

import jax
import jax.numpy as jnp
from jax import lax
from jax.experimental import pallas as pl
from jax.experimental.pallas import tpu as pltpu
from jax.experimental.pallas import tpu_sc as plsc

VOCAB = 100000
EMBED = 16
BATCH = 1024
CTX = 20

NC, NS = 2, 16
NW = NC * NS
ITEMS_PER_W = BATCH // NW
IDX_PER_W = ITEMS_PER_W * CTX
CHUNK = 128
NCHUNK = IDX_PER_W // CHUNK

VB = 1024
NBUF = 4
NFULL = VOCAB // VB
TAIL = VOCAB - NFULL * VB


def _sc_gather_mean(x3d, emb_table):
    mesh = plsc.VectorSubcoreMesh(core_axis_name="c", subcore_axis_name="s")

    @functools.partial(
        pl.kernel,
        out_type=jax.ShapeDtypeStruct((BATCH, EMBED), jnp.float32),
        mesh=mesh,
        scratch_types=[
            pltpu.VMEM((NCHUNK, CHUNK), jnp.int32),
            pltpu.VMEM((IDX_PER_W, EMBED), jnp.float32),
            pltpu.VMEM((ITEMS_PER_W, EMBED), jnp.float32),
            pltpu.SemaphoreType.DMA,
        ],
        compiler_params=pltpu.CompilerParams(use_tc_tiling_on_sc=False),
    )
    def k(x_hbm, tab_hbm, out_hbm, idx_v, rows_v, acc_v, sem):
        wid = lax.axis_index("s") * NC + lax.axis_index("c")
        pltpu.sync_copy(x_hbm.at[wid], idx_v)
        copies = []
        for j in range(NCHUNK):
            copies.append(pltpu.async_copy(
                tab_hbm.at[idx_v.at[j]],
                rows_v.at[pl.ds(j * CHUNK, CHUNK)],
                sem,
            ))
        for c in copies:
            c.wait()

        def body(i, _):
            base = i * CTX
            s = rows_v[base, :]
            for t in range(1, CTX):
                s = s + rows_v[base + t, :]
            acc_v[i, :] = s * (1.0 / CTX)
            return 0

        lax.fori_loop(0, ITEMS_PER_W, body, 0)
        pltpu.sync_copy(acc_v, out_hbm.at[pl.ds(wid * ITEMS_PER_W, ITEMS_PER_W)])

    return k(x3d, emb_table)


def _mm_body(wt_ref, avg_ref, b_ref, out_ref):
    out_ref[...] = lax.dot_general(
        wt_ref[...], avg_ref[...],
        (((0,), (1,)), ((), ())),
        preferred_element_type=jnp.float32,
    ) + jnp.transpose(b_ref[...])


def _tc_matmul_t(avg, Wt, b2):
    return pl.pallas_call(
        _mm_body,
        grid=(NFULL + 1,),
        in_specs=[
            pl.BlockSpec((EMBED, VB), lambda j: (0, j)),
            pl.BlockSpec((BATCH, EMBED), lambda j: (0, 0)),
            pl.BlockSpec((1, VB), lambda j: (0, j)),
        ],
        out_specs=pl.BlockSpec((VB, BATCH), lambda j: (j, 0)),
        out_shape=jax.ShapeDtypeStruct((VOCAB, BATCH), jnp.float32),
    )(Wt, avg, b2)


def kernel(x, emb_table, W, b):
    x3d = x.astype(jnp.int32).reshape(NW, NCHUNK, CHUNK)
    avg = _sc_gather_mean(x3d, emb_table)
    out_t = _tc_matmul_t(avg, W.T, b.reshape(1, VOCAB))
    return out_t.T


# speedup vs baseline: 1.1191x; 1.1191x over previous
import functools

import jax
import jax.numpy as jnp
from jax import lax
from jax.experimental import pallas as pl
from jax.experimental.pallas import tpu as pltpu
from jax.experimental.pallas import tpu_sc as plsc

VOCAB = 100000
EMBED = 16
BATCH = 1024
CTX = 20

NC, NS = 2, 16
NW = NC * NS
ITEMS_PER_W = BATCH // NW
IDX_PER_W = ITEMS_PER_W * CTX
CHUNK = 128
NCHUNK = IDX_PER_W // CHUNK

VB = 2048
NFULL = VOCAB // VB


def _sc_gather_mean(x3d, emb_table):
    mesh = plsc.VectorSubcoreMesh(core_axis_name="c", subcore_axis_name="s")

    @functools.partial(
        pl.kernel,
        out_type=jax.ShapeDtypeStruct((BATCH, EMBED), jnp.float32),
        mesh=mesh,
        scratch_types=[
            pltpu.VMEM((NCHUNK, CHUNK), jnp.int32),
            pltpu.VMEM((IDX_PER_W, EMBED), jnp.float32),
            pltpu.VMEM((ITEMS_PER_W, EMBED), jnp.float32),
            pltpu.SemaphoreType.DMA,
        ],
        compiler_params=pltpu.CompilerParams(use_tc_tiling_on_sc=False),
    )
    def k(x_hbm, tab_hbm, out_hbm, idx_v, rows_v, acc_v, sem):
        wid = lax.axis_index("s") * NC + lax.axis_index("c")
        pltpu.sync_copy(x_hbm.at[wid], idx_v)
        copies = []
        for j in range(NCHUNK):
            copies.append(pltpu.async_copy(
                tab_hbm.at[idx_v.at[j]],
                rows_v.at[pl.ds(j * CHUNK, CHUNK)],
                sem,
            ))
        for c in copies:
            c.wait()

        def body(i, _):
            base = i * CTX
            s = rows_v[base, :]
            for t in range(1, CTX):
                s = s + rows_v[base + t, :]
            acc_v[i, :] = s * (1.0 / CTX)
            return 0

        lax.fori_loop(0, ITEMS_PER_W, body, 0)
        pltpu.sync_copy(acc_v, out_hbm.at[pl.ds(wid * ITEMS_PER_W, ITEMS_PER_W)])

    return k(x3d, emb_table)


def _mm_body(wt_ref, avg_ref, b_ref, out_ref):
    out_ref[...] = lax.dot_general(
        wt_ref[...], avg_ref[...],
        (((0,), (1,)), ((), ())),
        preferred_element_type=jnp.float32,
    ) + jnp.transpose(b_ref[...])


def _tc_matmul_t(avg, Wt, b2):
    return pl.pallas_call(
        _mm_body,
        grid=(NFULL + 1,),
        in_specs=[
            pl.BlockSpec((EMBED, VB), lambda j: (0, j)),
            pl.BlockSpec((BATCH, EMBED), lambda j: (0, 0)),
            pl.BlockSpec((1, VB), lambda j: (0, j)),
        ],
        out_specs=pl.BlockSpec((VB, BATCH), lambda j: (j, 0)),
        out_shape=jax.ShapeDtypeStruct((VOCAB, BATCH), jnp.float32),
    )(Wt, avg, b2)


def kernel(x, emb_table, W, b):
    x3d = x.astype(jnp.int32).reshape(NW, NCHUNK, CHUNK)
    avg = _sc_gather_mean(x3d, emb_table)
    out_t = _tc_matmul_t(avg, W.T, b.reshape(1, VOCAB))
    return out_t.T
